# Initial kernel scaffold; baseline (speedup 1.0000x reference)
import functools
import jax
import jax.numpy as jnp
from jax import lax
from jax.experimental import pallas as pl
from jax.experimental.pallas import tpu as pltpu
from jax.experimental.pallas import tpu_sc as plsc

N = 10000
E = 320000
F_IN = 128
H = 8
C1 = 8
C2 = 16

NC = 2
NS = 16
L = 16
NW = NC * NS
EPT = E // NW
EB = 80
NB = EPT // EB
RPT = N // NS
ZR = 125


def _vgather(x, idx):
    dn = lax.GatherDimensionNumbers(
        offset_dims=(), collapsed_slice_dims=(0,), start_index_map=(0,))
    return lax.gather(x, idx[:, None], dn, (1,),
                      mode=lax.GatherScatterMode.PROMISE_IN_BOUNDS)


def _make_sc_layer(C):
    HC = H * C
    NV = HC // L
    if C == 16:
        scale_idx = [jnp.full((L,), v, jnp.int32) for v in range(NV)]
    else:
        scale_idx = [jnp.array([2 * v] * 8 + [2 * v + 1] * 8, jnp.int32)
                     for v in range(NV)]
    rot8 = jnp.array([8, 9, 10, 11, 12, 13, 14, 15] * 2, jnp.int32)
    mesh = plsc.VectorSubcoreMesh(core_axis_name="c", subcore_axis_name="s")

    def body(p_hbm, h_hbm, src_hbm, dst_hbm, out_hbm, s_hbm,
             src_v, dst_v, ps_v, pd_v, hr_v, msg_v, ex_v, z_v, zs_v,
             out_acc, s_acc, sem):
        c = lax.axis_index("c")
        s = lax.axis_index("s")
        wid = c * NS + s
        lane = lax.iota(jnp.int32, L)

        def zfill(r, carry):
            for v in range(NV):
                z_v[r, pl.ds(v * L, L)] = jnp.zeros((L,), jnp.float32)
            zs_v[r, :] = jnp.zeros((L,), jnp.float32)
            return carry
        lax.fori_loop(0, ZR, zfill, 0)
        row0 = s * RPT
        for q in range(RPT // ZR):
            pltpu.sync_copy(z_v, out_acc.at[pl.ds(row0 + q * ZR, ZR)])
            pltpu.sync_copy(zs_v, s_acc.at[pl.ds(row0 + q * ZR, ZR)])
        plsc.subcore_barrier()

        def batch(it, carry):
            base = wid * EPT + it * EB
            pltpu.sync_copy(src_hbm.at[pl.ds(base, EB)], src_v)
            pltpu.sync_copy(dst_hbm.at[pl.ds(base, EB)], dst_v)
            pltpu.async_copy(p_hbm.at[src_v], ps_v, sem).wait()
            pltpu.async_copy(p_hbm.at[dst_v], pd_v, sem).wait()
            pltpu.async_copy(h_hbm.at[src_v], hr_v, sem).wait()

            def edge(j, inner):
                ps = ps_v[j]
                pd = pd_v[j]
                e = ps + _vgather(pd, rot8)
                e = jnp.where(e > 0, e, 0.2 * e)
                ex = jnp.exp(e)
                ex_v[j] = jnp.where(lane < H, ex, 0.0)
                for v in range(NV):
                    sc = _vgather(ex, scale_idx[v])
                    msg_v[j, pl.ds(v * L, L)] = (
                        hr_v[j, pl.ds(v * L, L)] * sc)
                return inner
            lax.fori_loop(0, EB, edge, 0)

            pltpu.sync_copy(msg_v, out_acc.at[dst_v], add=True)
            pltpu.sync_copy(ex_v, s_acc.at[dst_v], add=True)
            return carry
        lax.fori_loop(0, NB, batch, 0)
        plsc.subcore_barrier()

        pltpu.sync_copy(out_acc.at[pl.ds(row0, RPT)],
                        out_hbm.at[c, pl.ds(row0, RPT)])
        pltpu.sync_copy(s_acc.at[pl.ds(row0, RPT)],
                        s_hbm.at[c, pl.ds(row0, RPT)])

    return pl.kernel(
        body, mesh=mesh,
        out_type=(jax.ShapeDtypeStruct((NC, N, HC), jnp.float32),
                  jax.ShapeDtypeStruct((NC, N, L), jnp.float32)),
        scratch_types=[
            pltpu.VMEM((EB,), jnp.int32),
            pltpu.VMEM((EB,), jnp.int32),
            pltpu.VMEM((EB, L), jnp.float32),
            pltpu.VMEM((EB, L), jnp.float32),
            pltpu.VMEM((EB, HC), jnp.float32),
            pltpu.VMEM((EB, HC), jnp.float32),
            pltpu.VMEM((EB, L), jnp.float32),
            pltpu.VMEM((ZR, HC), jnp.float32),
            pltpu.VMEM((ZR, L), jnp.float32),
            pltpu.VMEM_SHARED((N, HC), jnp.float32),
            pltpu.VMEM_SHARED((N, L), jnp.float32),
            pltpu.SemaphoreType.DMA,
        ],
        name=f"gat_edges_c{C}",
    )


_sc_layer1 = _make_sc_layer(C1)
_sc_layer2 = _make_sc_layer(C2)


def _tc_embed(x, W, Apack):
    n, hc = x.shape[0], W.shape[1]

    def body(x_ref, w_ref, a_ref, h_ref, p_ref):
        h = jnp.dot(x_ref[...], w_ref[...],
                    preferred_element_type=jnp.float32)
        h_ref[...] = h
        p_ref[...] = jnp.dot(h, a_ref[...],
                             preferred_element_type=jnp.float32)

    return pl.pallas_call(
        body,
        out_shape=(jax.ShapeDtypeStruct((n, hc), jnp.float32),
                   jax.ShapeDtypeStruct((n, 2 * H), jnp.float32)),
        name="gat_embed",
    )(x, W, Apack)


def _tc_mid(op, sp, b1, W2, A2pack, Rexp):
    def body(op_ref, sp_ref, b_ref, w_ref, a_ref, r_ref, h_ref, p_ref):
        ou = op_ref[0] + op_ref[1]
        sv = sp_ref[0] + sp_ref[1]
        scale = jnp.dot(1.0 / (sv + 1e-16), r_ref[...],
                        preferred_element_type=jnp.float32)
        u = ou * scale + b_ref[...]
        a = jnp.where(u > 0, u, jnp.exp(jnp.minimum(u, 0.0)) - 1.0)
        h2 = jnp.dot(a, w_ref[...], preferred_element_type=jnp.float32)
        h_ref[...] = h2
        p_ref[...] = jnp.dot(h2, a_ref[...],
                             preferred_element_type=jnp.float32)

    return pl.pallas_call(
        body,
        out_shape=(jax.ShapeDtypeStruct((N, H * C2), jnp.float32),
                   jax.ShapeDtypeStruct((N, 2 * H), jnp.float32)),
        name="gat_mid",
    )(op, sp, b1, W2, A2pack, Rexp)


def _tc_final(op, sp, b2, Rexp):
    def body(op_ref, sp_ref, b_ref, r_ref, o_ref):
        ou = op_ref[0] + op_ref[1]
        sv = sp_ref[0] + sp_ref[1]
        scale = jnp.dot(1.0 / (sv + 1e-16), r_ref[...],
                        preferred_element_type=jnp.float32)
        y = ou * scale + b_ref[...]
        m = jnp.max(y, axis=1, keepdims=True)
        z = y - m
        o_ref[...] = z - jnp.log(jnp.sum(jnp.exp(z), axis=1, keepdims=True))

    return pl.pallas_call(
        body,
        out_shape=jax.ShapeDtypeStruct((N, H * C2), jnp.float32),
        name="gat_final",
    )(op, sp, b2, Rexp)


def _packs(a_src, a_dst, C):
    eye = jnp.eye(H, dtype=jnp.float32)
    Ms = jnp.einsum("hc,hg->hcg", a_src, eye).reshape(H * C, H)
    Md = jnp.einsum("hc,hg->hcg", a_dst, eye).reshape(H * C, H)
    return jnp.concatenate([Ms, Md], axis=1)


def _rexp(C):
    top = jnp.kron(jnp.eye(H, dtype=jnp.float32),
                   jnp.ones((1, C), dtype=jnp.float32))
    return jnp.concatenate(
        [top, jnp.zeros((L - H, H * C), dtype=jnp.float32)], axis=0)


def kernel(x, edge_index, W1, a1_src, a1_dst, b1, W2, a2_src, a2_dst, b2):
    src = edge_index[0]
    dst = edge_index[1]

    h1, p1 = _tc_embed(x, W1, _packs(a1_src, a1_dst, C1))
    op1, sp1 = _sc_layer1(p1, h1, src, dst)
    h2, p2 = _tc_mid(op1, sp1, b1.reshape(1, H * C1), W2,
                     _packs(a2_src, a2_dst, C2), _rexp(C1))
    op2, sp2 = _sc_layer2(p2, h2, src, dst)
    return _tc_final(op2, sp2, b2.reshape(1, H * C2), _rexp(C2))


# speedup vs baseline: 43.6922x; 43.6922x over previous
import functools
import numpy as np
import jax
import jax.numpy as jnp
from jax import lax
from jax.experimental import pallas as pl
from jax.experimental.pallas import tpu as pltpu
from jax.experimental.pallas import tpu_sc as plsc

N = 10000
E = 320000
F_IN = 128
H = 8
C1 = 8
C2 = 16

NC = 2
NS = 16
L = 16
NW = NC * NS
EPT = E // NW
EB = 80
NB = EPT // EB
WB = 624
ZR = 104


def _vgather(x, idx):
    dn = lax.GatherDimensionNumbers(
        offset_dims=(), collapsed_slice_dims=(0,), start_index_map=(0,))
    return lax.gather(x, idx[:, None], dn, (1,),
                      mode=lax.GatherScatterMode.PROMISE_IN_BOUNDS)


def _make_sc_layer(C):
    HC = H * C
    NV = HC // L
    mesh = plsc.VectorSubcoreMesh(core_axis_name="c", subcore_axis_name="s")

    def body(p_hbm, h_hbm, src_hbm, dst_hbm, out_hbm, s_hbm,
             src_v, dst_v, ps_v, pd_v, hr_v, msg_v, ex_v, z_v, zs_v,
             out_acc, s_acc, sem):
        c = lax.axis_index("c")
        s = lax.axis_index("s")
        wid = c * NS + s
        lane = lax.iota(jnp.int32, L)
        rot8 = (lane & 7) + 8
        if C == 16:
            scale_idx = [lane * 0 + v for v in range(NV)]
        else:
            scale_idx = [2 * v + (lane >> 3) for v in range(NV)]

        def zfill(r, carry):
            for v in range(NV):
                z_v[r, pl.ds(v * L, L)] = jnp.zeros((L,), jnp.float32)
            zs_v[r, :] = jnp.zeros((L,), jnp.float32)
            return carry
        lax.fori_loop(0, ZR, zfill, 0)
        row0 = s * WB
        for q in range(WB // ZR):
            pltpu.sync_copy(z_v, out_acc.at[pl.ds(row0 + q * ZR, ZR)])
            pltpu.sync_copy(zs_v, s_acc.at[pl.ds(row0 + q * ZR, ZR)])

        @pl.when(s == NS - 1)
        def _():
            pltpu.sync_copy(z_v.at[pl.ds(0, N - NS * WB)],
                            out_acc.at[pl.ds(NS * WB, N - NS * WB)])
            pltpu.sync_copy(zs_v.at[pl.ds(0, N - NS * WB)],
                            s_acc.at[pl.ds(NS * WB, N - NS * WB)])
        plsc.subcore_barrier()

        def batch(it, carry):
            base = wid * EPT + it * EB
            pltpu.sync_copy(src_hbm.at[pl.ds(base, EB)], src_v)
            pltpu.sync_copy(dst_hbm.at[pl.ds(base, EB)], dst_v)
            pltpu.async_copy(p_hbm.at[src_v], ps_v, sem).wait()
            pltpu.async_copy(p_hbm.at[dst_v], pd_v, sem).wait()
            pltpu.async_copy(h_hbm.at[src_v], hr_v, sem).wait()

            def edge(j, inner):
                ps = ps_v[j]
                pd = pd_v[j]
                e = ps + _vgather(pd, rot8)
                e = jnp.where(e > 0, e, 0.2 * e)
                ex = jnp.exp(e)
                ex_v[j] = jnp.where(lane < H, ex, 0.0)
                for v in range(NV):
                    sc = _vgather(ex, scale_idx[v])
                    msg_v[j, pl.ds(v * L, L)] = (
                        hr_v[j, pl.ds(v * L, L)] * sc)
                return inner
            lax.fori_loop(0, EB, edge, 0)

            pltpu.sync_copy(msg_v, out_acc.at[dst_v], add=True)
            pltpu.sync_copy(ex_v, s_acc.at[dst_v], add=True)
            return carry
        lax.fori_loop(0, NB, batch, 0)
        plsc.subcore_barrier()

        pltpu.sync_copy(out_acc.at[pl.ds(row0, WB)],
                        out_hbm.at[c, pl.ds(row0, WB)])
        pltpu.sync_copy(s_acc.at[pl.ds(row0, WB)],
                        s_hbm.at[c, pl.ds(row0, WB)])

        @pl.when(s == NS - 1)
        def _():
            pltpu.sync_copy(out_acc.at[pl.ds(NS * WB, N - NS * WB)],
                            out_hbm.at[c, pl.ds(NS * WB, N - NS * WB)])
            pltpu.sync_copy(s_acc.at[pl.ds(NS * WB, N - NS * WB)],
                            s_hbm.at[c, pl.ds(NS * WB, N - NS * WB)])

    return pl.kernel(
        body, mesh=mesh,
        compiler_params=pltpu.CompilerParams(use_tc_tiling_on_sc=False),
        out_type=(jax.ShapeDtypeStruct((NC, N, HC), jnp.float32),
                  jax.ShapeDtypeStruct((NC, N, L), jnp.float32)),
        scratch_types=[
            pltpu.VMEM((EB,), jnp.int32),
            pltpu.VMEM((EB,), jnp.int32),
            pltpu.VMEM((EB, L), jnp.float32),
            pltpu.VMEM((EB, L), jnp.float32),
            pltpu.VMEM((EB, HC), jnp.float32),
            pltpu.VMEM((EB, HC), jnp.float32),
            pltpu.VMEM((EB, L), jnp.float32),
            pltpu.VMEM((ZR, HC), jnp.float32),
            pltpu.VMEM((ZR, L), jnp.float32),
            pltpu.VMEM_SHARED((N, HC), jnp.float32),
            pltpu.VMEM_SHARED((N, L), jnp.float32),
            pltpu.SemaphoreType.DMA,
        ],
        name=f"gat_edges_c{C}",
    )


_sc_layer1 = _make_sc_layer(C1)
_sc_layer2 = _make_sc_layer(C2)


def _tc_embed(x, W, Apack):
    n, hc = x.shape[0], W.shape[1]

    def body(x_ref, w_ref, a_ref, h_ref, p_ref):
        h = jnp.dot(x_ref[...], w_ref[...],
                    preferred_element_type=jnp.float32)
        h_ref[...] = h
        p_ref[...] = jnp.dot(h, a_ref[...],
                             preferred_element_type=jnp.float32)

    return pl.pallas_call(
        body,
        out_shape=(jax.ShapeDtypeStruct((n, hc), jnp.float32),
                   jax.ShapeDtypeStruct((n, 2 * H), jnp.float32)),
        name="gat_embed",
    )(x, W, Apack)


def _tc_mid(op, sp, b1, W2, A2pack, Rexp):
    def body(op_ref, sp_ref, b_ref, w_ref, a_ref, r_ref, h_ref, p_ref):
        ou = op_ref[0] + op_ref[1]
        sv = sp_ref[0] + sp_ref[1]
        scale = jnp.dot(1.0 / (sv + 1e-16), r_ref[...],
                        preferred_element_type=jnp.float32)
        u = ou * scale + b_ref[...]
        a = jnp.where(u > 0, u, jnp.exp(jnp.minimum(u, 0.0)) - 1.0)
        h2 = jnp.dot(a, w_ref[...], preferred_element_type=jnp.float32)
        h_ref[...] = h2
        p_ref[...] = jnp.dot(h2, a_ref[...],
                             preferred_element_type=jnp.float32)

    return pl.pallas_call(
        body,
        out_shape=(jax.ShapeDtypeStruct((N, H * C2), jnp.float32),
                   jax.ShapeDtypeStruct((N, 2 * H), jnp.float32)),
        name="gat_mid",
    )(op, sp, b1, W2, A2pack, Rexp)


def _tc_final(op, sp, b2, Rexp):
    def body(op_ref, sp_ref, b_ref, r_ref, o_ref):
        ou = op_ref[0] + op_ref[1]
        sv = sp_ref[0] + sp_ref[1]
        scale = jnp.dot(1.0 / (sv + 1e-16), r_ref[...],
                        preferred_element_type=jnp.float32)
        y = ou * scale + b_ref[...]
        m = jnp.max(y, axis=1, keepdims=True)
        z = y - m
        o_ref[...] = z - jnp.log(jnp.sum(jnp.exp(z), axis=1, keepdims=True))

    return pl.pallas_call(
        body,
        out_shape=jax.ShapeDtypeStruct((N, H * C2), jnp.float32),
        name="gat_final",
    )(op, sp, b2, Rexp)


def _packs(a_src, a_dst, C):
    eye = jnp.eye(H, dtype=jnp.float32)
    Ms = jnp.einsum("hc,hg->hcg", a_src, eye).reshape(H * C, H)
    Md = jnp.einsum("hc,hg->hcg", a_dst, eye).reshape(H * C, H)
    return jnp.concatenate([Ms, Md], axis=1)


def _rexp(C):
    top = jnp.kron(jnp.eye(H, dtype=jnp.float32),
                   jnp.ones((1, C), dtype=jnp.float32))
    return jnp.concatenate(
        [top, jnp.zeros((L - H, H * C), dtype=jnp.float32)], axis=0)


def kernel(x, edge_index, W1, a1_src, a1_dst, b1, W2, a2_src, a2_dst, b2):
    src = edge_index[0]
    dst = edge_index[1]

    h1, p1 = _tc_embed(x, W1, _packs(a1_src, a1_dst, C1))
    op1, sp1 = _sc_layer1(p1, h1, src, dst)
    h2, p2 = _tc_mid(op1, sp1, b1.reshape(1, H * C1), W2,
                     _packs(a2_src, a2_dst, C2), _rexp(C1))
    op2, sp2 = _sc_layer2(p2, h2, src, dst)
    return _tc_final(op2, sp2, b2.reshape(1, H * C2), _rexp(C2))


# speedup vs baseline: 69.0395x; 1.5801x over previous
import functools
import numpy as np
import jax
import jax.numpy as jnp
from jax import lax
from jax.experimental import pallas as pl
from jax.experimental.pallas import tpu as pltpu
from jax.experimental.pallas import tpu_sc as plsc

N = 10000
E = 320000
F_IN = 128
H = 8
C1 = 8
C2 = 16

NC = 2
NS = 16
L = 16
NW = NC * NS
EPT = E // NW
EB = 80
NB = EPT // EB
WB = 624
ZR = 104


def _vgather(x, idx):
    dn = lax.GatherDimensionNumbers(
        offset_dims=(), collapsed_slice_dims=(0,), start_index_map=(0,))
    return lax.gather(x, idx[:, None], dn, (1,),
                      mode=lax.GatherScatterMode.PROMISE_IN_BOUNDS)


def _make_sc_layer(C, HO):
    HS = 64 // C
    HC = HS * C
    NV = HC // L
    mesh = plsc.VectorSubcoreMesh(core_axis_name="c", subcore_axis_name="s")

    def body(p_hbm, h_hbm, src_hbm, dst_hbm, out_hbm, s_hbm,
             src2_v, dst2_v, ps0, ps1, pd0, pd1, hr0, hr1,
             msg0, msg1, ex0, ex1, z_v, zs_v,
             out_acc, s_acc, gsem0, gsem1, ssem0, ssem1):
        c = lax.axis_index("c")
        s = lax.axis_index("s")
        wid = c * NS + s
        lane = lax.iota(jnp.int32, L)
        rot8 = (lane & 7) + 8
        if C == 16:
            scale_idx = [lane * 0 + (HO + v) for v in range(NV)]
        else:
            scale_idx = [2 * v + (lane >> 3) for v in range(NV)]

        bufs = [(ps0, pd0, hr0, msg0, ex0, gsem0, ssem0),
                (ps1, pd1, hr1, msg1, ex1, gsem1, ssem1)]

        pltpu.sync_copy(src_hbm.at[pl.ds(wid * NB, NB)], src2_v)
        pltpu.sync_copy(dst_hbm.at[pl.ds(wid * NB, NB)], dst2_v)

        def issue_gathers(it, b):
            ps, pd, hr, _, _, gsem, _ = bufs[b]
            pltpu.async_copy(p_hbm.at[src2_v.at[it]], ps, gsem)
            pltpu.async_copy(p_hbm.at[dst2_v.at[it]], pd, gsem)
            pltpu.async_copy(h_hbm.at[src2_v.at[it]], hr, gsem)

        def drain_gathers(b):
            ps, pd, hr, _, _, gsem, _ = bufs[b]
            pltpu.make_async_copy(p_hbm.at[pl.ds(0, EB)], ps, gsem).wait()
            pltpu.make_async_copy(p_hbm.at[pl.ds(0, EB)], pd, gsem).wait()
            pltpu.make_async_copy(h_hbm.at[pl.ds(0, EB)], hr, gsem).wait()

        def issue_scatters(it, b):
            _, _, _, msg, ex, _, ssem = bufs[b]
            pltpu.async_copy(msg, out_acc.at[dst2_v.at[it]], ssem, add=True)
            pltpu.async_copy(ex, s_acc.at[dst2_v.at[it]], ssem, add=True)

        def drain_scatters(b):
            _, _, _, msg, ex, _, ssem = bufs[b]
            pltpu.make_async_copy(msg, out_acc.at[pl.ds(0, EB)], ssem).wait()
            pltpu.make_async_copy(ex, s_acc.at[pl.ds(0, EB)], ssem).wait()

        def compute(b):
            ps_v, pd_v, hr_v, msg_v, ex_v, _, _ = bufs[b]

            def edge(j, inner):
                ps = ps_v[j]
                pd = pd_v[j]
                e = ps + _vgather(pd, rot8)
                e = jnp.where(e > 0, e, 0.2 * e)
                ex = jnp.exp(e)
                ex_v[j] = jnp.where((lane >= HO) & (lane < HO + HS), ex, 0.0)
                for v in range(NV):
                    sc = _vgather(ex, scale_idx[v])
                    msg_v[j, pl.ds(v * L, L)] = (
                        hr_v[j, pl.ds(v * L, L)] * sc)
                return inner
            lax.fori_loop(0, EB, edge, 0)

        def step(it, b, has_next, drain_scatter):
            if has_next:
                issue_gathers(it + 1, 1 - b)
            drain_gathers(b)
            if drain_scatter:
                drain_scatters(b)
            compute(b)
            issue_scatters(it, b)

        issue_gathers(0, 0)

        def zfill(r, carry):
            for v in range(NV):
                z_v[r, pl.ds(v * L, L)] = jnp.zeros((L,), jnp.float32)
            zs_v[r, :] = jnp.zeros((L,), jnp.float32)
            return carry
        lax.fori_loop(0, ZR, zfill, 0)
        row0 = s * WB
        for q in range(WB // ZR):
            pltpu.sync_copy(z_v, out_acc.at[pl.ds(row0 + q * ZR, ZR)])
            pltpu.sync_copy(zs_v, s_acc.at[pl.ds(row0 + q * ZR, ZR)])

        @pl.when(s == NS - 1)
        def _():
            pltpu.sync_copy(z_v.at[pl.ds(0, N - NS * WB)],
                            out_acc.at[pl.ds(NS * WB, N - NS * WB)])
            pltpu.sync_copy(zs_v.at[pl.ds(0, N - NS * WB)],
                            s_acc.at[pl.ds(NS * WB, N - NS * WB)])
        plsc.subcore_barrier()

        step(0, 0, True, False)
        step(1, 1, True, False)

        def pair(k, carry):
            step(2 * k, 0, True, True)
            step(2 * k + 1, 1, True, True)
            return carry
        lax.fori_loop(1, (NB - 1) // 2, pair, 0)
        step(NB - 1, 0, False, True)
        drain_scatters(1)
        drain_scatters(0)
        plsc.subcore_barrier()

        pltpu.sync_copy(out_acc.at[pl.ds(row0, WB)],
                        out_hbm.at[c, pl.ds(row0, WB)])
        pltpu.sync_copy(s_acc.at[pl.ds(row0, WB)],
                        s_hbm.at[c, pl.ds(row0, WB)])

        @pl.when(s == NS - 1)
        def _():
            pltpu.sync_copy(out_acc.at[pl.ds(NS * WB, N - NS * WB)],
                            out_hbm.at[c, pl.ds(NS * WB, N - NS * WB)])
            pltpu.sync_copy(s_acc.at[pl.ds(NS * WB, N - NS * WB)],
                            s_hbm.at[c, pl.ds(NS * WB, N - NS * WB)])

    return pl.kernel(
        body, mesh=mesh,
        compiler_params=pltpu.CompilerParams(use_tc_tiling_on_sc=False),
        out_type=(jax.ShapeDtypeStruct((NC, N, HC), jnp.float32),
                  jax.ShapeDtypeStruct((NC, N, L), jnp.float32)),
        scratch_types=[
            pltpu.VMEM((NB, EB), jnp.int32),
            pltpu.VMEM((NB, EB), jnp.int32),
            pltpu.VMEM((EB, L), jnp.float32),
            pltpu.VMEM((EB, L), jnp.float32),
            pltpu.VMEM((EB, L), jnp.float32),
            pltpu.VMEM((EB, L), jnp.float32),
            pltpu.VMEM((EB, HC), jnp.float32),
            pltpu.VMEM((EB, HC), jnp.float32),
            pltpu.VMEM((EB, HC), jnp.float32),
            pltpu.VMEM((EB, HC), jnp.float32),
            pltpu.VMEM((EB, L), jnp.float32),
            pltpu.VMEM((EB, L), jnp.float32),
            pltpu.VMEM((ZR, HC), jnp.float32),
            pltpu.VMEM((ZR, L), jnp.float32),
            pltpu.VMEM_SHARED((N, HC), jnp.float32),
            pltpu.VMEM_SHARED((N, L), jnp.float32),
            pltpu.SemaphoreType.DMA,
            pltpu.SemaphoreType.DMA,
            pltpu.SemaphoreType.DMA,
            pltpu.SemaphoreType.DMA,
        ],
        name=f"gat_edges_c{C}h{HO}",
    )


_sc_layer1 = _make_sc_layer(C1, 0)
_sc_layer2a = _make_sc_layer(C2, 0)
_sc_layer2b = _make_sc_layer(C2, 4)


def _tc_embed(x, W, Apack):
    n, hc = x.shape[0], W.shape[1]

    def body(x_ref, w_ref, a_ref, h_ref, p_ref):
        h = jnp.dot(x_ref[...], w_ref[...],
                    preferred_element_type=jnp.float32)
        h_ref[...] = h
        p_ref[...] = jnp.dot(h, a_ref[...],
                             preferred_element_type=jnp.float32)

    return pl.pallas_call(
        body,
        out_shape=(jax.ShapeDtypeStruct((n, hc), jnp.float32),
                   jax.ShapeDtypeStruct((n, 2 * H), jnp.float32)),
        name="gat_embed",
    )(x, W, Apack)


def _tc_mid(op, sp, b1, W2, A2pack, Rexp):
    def body(op_ref, sp_ref, b_ref, w_ref, a_ref, r_ref,
             ha_ref, hb_ref, p_ref):
        ou = op_ref[0] + op_ref[1]
        sv = sp_ref[0] + sp_ref[1]
        scale = jnp.dot(1.0 / (sv + 1e-16), r_ref[...],
                        preferred_element_type=jnp.float32)
        u = ou * scale + b_ref[...]
        a = jnp.where(u > 0, u, jnp.exp(jnp.minimum(u, 0.0)) - 1.0)
        h2 = jnp.dot(a, w_ref[...], preferred_element_type=jnp.float32)
        ha_ref[...] = h2[:, : H * C2 // 2]
        hb_ref[...] = h2[:, H * C2 // 2:]
        p_ref[...] = jnp.dot(h2, a_ref[...],
                             preferred_element_type=jnp.float32)

    return pl.pallas_call(
        body,
        out_shape=(jax.ShapeDtypeStruct((N, H * C2 // 2), jnp.float32),
                   jax.ShapeDtypeStruct((N, H * C2 // 2), jnp.float32),
                   jax.ShapeDtypeStruct((N, 2 * H), jnp.float32)),
        name="gat_mid",
    )(op, sp, b1, W2, A2pack, Rexp)


def _tc_final(opa, opb, spa, spb, b2, Rexp):
    def body(opa_ref, opb_ref, spa_ref, spb_ref, b_ref, r_ref, o_ref):
        ou = jnp.concatenate(
            [opa_ref[0] + opa_ref[1], opb_ref[0] + opb_ref[1]], axis=1)
        sv = spa_ref[0] + spa_ref[1] + spb_ref[0] + spb_ref[1]
        scale = jnp.dot(1.0 / (sv + 1e-16), r_ref[...],
                        preferred_element_type=jnp.float32)
        y = ou * scale + b_ref[...]
        m = jnp.max(y, axis=1, keepdims=True)
        z = y - m
        o_ref[...] = z - jnp.log(jnp.sum(jnp.exp(z), axis=1, keepdims=True))

    return pl.pallas_call(
        body,
        out_shape=jax.ShapeDtypeStruct((N, H * C2), jnp.float32),
        name="gat_final",
    )(opa, opb, spa, spb, b2, Rexp)


def _packs(a_src, a_dst, C):
    eye = jnp.eye(H, dtype=jnp.float32)
    Ms = jnp.einsum("hc,hg->hcg", a_src, eye).reshape(H * C, H)
    Md = jnp.einsum("hc,hg->hcg", a_dst, eye).reshape(H * C, H)
    return jnp.concatenate([Ms, Md], axis=1)


def _rexp(C):
    top = jnp.kron(jnp.eye(H, dtype=jnp.float32),
                   jnp.ones((1, C), dtype=jnp.float32))
    return jnp.concatenate(
        [top, jnp.zeros((L - H, H * C), dtype=jnp.float32)], axis=0)


def kernel(x, edge_index, W1, a1_src, a1_dst, b1, W2, a2_src, a2_dst, b2):
    src = edge_index[0].reshape(E // EB, EB)
    dst = edge_index[1].reshape(E // EB, EB)

    h1, p1 = _tc_embed(x, W1, _packs(a1_src, a1_dst, C1))
    op1, sp1 = _sc_layer1(p1, h1, src, dst)
    h2a, h2b, p2 = _tc_mid(op1, sp1, b1.reshape(1, H * C1), W2,
                           _packs(a2_src, a2_dst, C2), _rexp(C1))
    op2a, sp2a = _sc_layer2a(p2, h2a, src, dst)
    op2b, sp2b = _sc_layer2b(p2, h2b, src, dst)
    return _tc_final(op2a, op2b, sp2a, sp2b,
                     b2.reshape(1, H * C2), _rexp(C2))


# speedup vs baseline: 184.6542x; 2.6746x over previous
import functools
import numpy as np
import jax
import jax.numpy as jnp
from jax import lax
from jax.experimental import pallas as pl
from jax.experimental.pallas import tpu as pltpu
from jax.experimental.pallas import tpu_sc as plsc

N = 10000
E = 320000
F_IN = 128
H = 8
C1 = 8
C2 = 16

NC = 2
NS = 16
L = 16
NW = NC * NS
EPT = E // NW
EB = 80
NB = EPT // EB
WB = 624
ZR = 104


def _vgather(x, idx):
    dn = lax.GatherDimensionNumbers(
        offset_dims=(), collapsed_slice_dims=(0,), start_index_map=(0,))
    return lax.gather(x, idx[:, None], dn, (1,),
                      mode=lax.GatherScatterMode.PROMISE_IN_BOUNDS)


def _make_sc_layer(C, HO):
    HS = 64 // C
    HC = HS * C
    NV = HC // L
    mesh = plsc.VectorSubcoreMesh(core_axis_name="c", subcore_axis_name="s")

    def body(p_hbm, h_hbm, src_hbm, dst_hbm, out_hbm, s_hbm,
             src2_v, dst2_v, ps0, ps1, pd0, pd1, hr0, hr1,
             msg0, msg1, ex0, ex1, z_v, zs_v,
             out_acc, s_acc, gsem0, gsem1, ssem0, ssem1):
        c = lax.axis_index("c")
        s = lax.axis_index("s")
        wid = c * NS + s
        lane = lax.iota(jnp.int32, L)
        rot8 = (lane & 7) + 8
        if C == 16:
            scale_idx = [lane * 0 + (HO + v) for v in range(NV)]
        else:
            scale_idx = [2 * v + (lane >> 3) for v in range(NV)]

        bufs = [(ps0, pd0, hr0, msg0, ex0, gsem0, ssem0),
                (ps1, pd1, hr1, msg1, ex1, gsem1, ssem1)]

        pltpu.sync_copy(src_hbm.at[pl.ds(wid * NB, NB)], src2_v)
        pltpu.sync_copy(dst_hbm.at[pl.ds(wid * NB, NB)], dst2_v)

        def issue_gathers(it, b):
            ps, pd, hr, _, _, gsem, _ = bufs[b]
            pltpu.async_copy(p_hbm.at[src2_v.at[it]], ps, gsem)
            pltpu.async_copy(p_hbm.at[dst2_v.at[it]], pd, gsem)
            pltpu.async_copy(h_hbm.at[src2_v.at[it]], hr, gsem)

        def drain_gathers(b):
            ps, pd, hr, _, _, gsem, _ = bufs[b]
            pltpu.make_async_copy(p_hbm.at[pl.ds(0, EB)], ps, gsem).wait()
            pltpu.make_async_copy(p_hbm.at[pl.ds(0, EB)], pd, gsem).wait()
            pltpu.make_async_copy(h_hbm.at[pl.ds(0, EB)], hr, gsem).wait()

        def issue_scatters(it, b):
            _, _, _, msg, ex, _, ssem = bufs[b]
            pltpu.async_copy(msg, out_acc.at[dst2_v.at[it]], ssem, add=True)
            pltpu.async_copy(ex, s_acc.at[dst2_v.at[it]], ssem, add=True)

        def drain_scatters(b):
            _, _, _, msg, ex, _, ssem = bufs[b]
            pltpu.make_async_copy(msg, out_acc.at[pl.ds(0, EB)], ssem).wait()
            pltpu.make_async_copy(ex, s_acc.at[pl.ds(0, EB)], ssem).wait()

        def compute(b):
            ps_v, pd_v, hr_v, msg_v, ex_v, _, _ = bufs[b]

            @plsc.parallel_loop(0, EB, 1, unroll=4)
            def edge(j):
                ps = ps_v[j]
                pd = pd_v[j]
                e = ps + _vgather(pd, rot8)
                e = jnp.where(e > 0, e, 0.2 * e)
                ex = jnp.exp(e)
                ex_v[j] = jnp.where((lane >= HO) & (lane < HO + HS), ex, 0.0)
                for v in range(NV):
                    sc = _vgather(ex, scale_idx[v])
                    msg_v[j, pl.ds(v * L, L)] = (
                        hr_v[j, pl.ds(v * L, L)] * sc)

        def step(it, b, has_next, drain_scatter):
            if has_next:
                issue_gathers(it + 1, 1 - b)
            drain_gathers(b)
            if drain_scatter:
                drain_scatters(b)
            compute(b)
            issue_scatters(it, b)

        issue_gathers(0, 0)

        def zfill(r, carry):
            for v in range(NV):
                z_v[r, pl.ds(v * L, L)] = jnp.zeros((L,), jnp.float32)
            zs_v[r, :] = jnp.zeros((L,), jnp.float32)
            return carry
        lax.fori_loop(0, ZR, zfill, 0)
        row0 = s * WB
        for q in range(WB // ZR):
            pltpu.sync_copy(z_v, out_acc.at[pl.ds(row0 + q * ZR, ZR)])
            pltpu.sync_copy(zs_v, s_acc.at[pl.ds(row0 + q * ZR, ZR)])

        @pl.when(s == NS - 1)
        def _():
            pltpu.sync_copy(z_v.at[pl.ds(0, N - NS * WB)],
                            out_acc.at[pl.ds(NS * WB, N - NS * WB)])
            pltpu.sync_copy(zs_v.at[pl.ds(0, N - NS * WB)],
                            s_acc.at[pl.ds(NS * WB, N - NS * WB)])
        plsc.subcore_barrier()

        step(0, 0, True, False)
        step(1, 1, True, False)

        def pair(k, carry):
            step(2 * k, 0, True, True)
            step(2 * k + 1, 1, True, True)
            return carry
        lax.fori_loop(1, (NB - 1) // 2, pair, 0)
        step(NB - 1, 0, False, True)
        drain_scatters(1)
        drain_scatters(0)
        plsc.subcore_barrier()

        pltpu.sync_copy(out_acc.at[pl.ds(row0, WB)],
                        out_hbm.at[c, pl.ds(row0, WB)])
        pltpu.sync_copy(s_acc.at[pl.ds(row0, WB)],
                        s_hbm.at[c, pl.ds(row0, WB)])

        @pl.when(s == NS - 1)
        def _():
            pltpu.sync_copy(out_acc.at[pl.ds(NS * WB, N - NS * WB)],
                            out_hbm.at[c, pl.ds(NS * WB, N - NS * WB)])
            pltpu.sync_copy(s_acc.at[pl.ds(NS * WB, N - NS * WB)],
                            s_hbm.at[c, pl.ds(NS * WB, N - NS * WB)])

    return pl.kernel(
        body, mesh=mesh,
        compiler_params=pltpu.CompilerParams(use_tc_tiling_on_sc=False),
        out_type=(jax.ShapeDtypeStruct((NC, N, HC), jnp.float32),
                  jax.ShapeDtypeStruct((NC, N, L), jnp.float32)),
        scratch_types=[
            pltpu.VMEM((NB, EB), jnp.int32),
            pltpu.VMEM((NB, EB), jnp.int32),
            pltpu.VMEM((EB, L), jnp.float32),
            pltpu.VMEM((EB, L), jnp.float32),
            pltpu.VMEM((EB, L), jnp.float32),
            pltpu.VMEM((EB, L), jnp.float32),
            pltpu.VMEM((EB, HC), jnp.float32),
            pltpu.VMEM((EB, HC), jnp.float32),
            pltpu.VMEM((EB, HC), jnp.float32),
            pltpu.VMEM((EB, HC), jnp.float32),
            pltpu.VMEM((EB, L), jnp.float32),
            pltpu.VMEM((EB, L), jnp.float32),
            pltpu.VMEM((ZR, HC), jnp.float32),
            pltpu.VMEM((ZR, L), jnp.float32),
            pltpu.VMEM_SHARED((N, HC), jnp.float32),
            pltpu.VMEM_SHARED((N, L), jnp.float32),
            pltpu.SemaphoreType.DMA,
            pltpu.SemaphoreType.DMA,
            pltpu.SemaphoreType.DMA,
            pltpu.SemaphoreType.DMA,
        ],
        name=f"gat_edges_c{C}h{HO}",
    )


_sc_layer1 = _make_sc_layer(C1, 0)
_sc_layer2a = _make_sc_layer(C2, 0)
_sc_layer2b = _make_sc_layer(C2, 4)


def _tc_embed(x, W, Apack):
    n, hc = x.shape[0], W.shape[1]

    def body(x_ref, w_ref, a_ref, h_ref, p_ref):
        h = jnp.dot(x_ref[...], w_ref[...],
                    preferred_element_type=jnp.float32)
        h_ref[...] = h
        p_ref[...] = jnp.dot(h, a_ref[...],
                             preferred_element_type=jnp.float32)

    return pl.pallas_call(
        body,
        out_shape=(jax.ShapeDtypeStruct((n, hc), jnp.float32),
                   jax.ShapeDtypeStruct((n, 2 * H), jnp.float32)),
        name="gat_embed",
    )(x, W, Apack)


def _tc_mid(op, sp, b1, W2, A2pack, Rexp):
    def body(op_ref, sp_ref, b_ref, w_ref, a_ref, r_ref,
             ha_ref, hb_ref, p_ref):
        ou = op_ref[0] + op_ref[1]
        sv = sp_ref[0] + sp_ref[1]
        scale = jnp.dot(1.0 / (sv + 1e-16), r_ref[...],
                        preferred_element_type=jnp.float32)
        u = ou * scale + b_ref[...]
        a = jnp.where(u > 0, u, jnp.exp(jnp.minimum(u, 0.0)) - 1.0)
        h2 = jnp.dot(a, w_ref[...], preferred_element_type=jnp.float32)
        ha_ref[...] = h2[:, : H * C2 // 2]
        hb_ref[...] = h2[:, H * C2 // 2:]
        p_ref[...] = jnp.dot(h2, a_ref[...],
                             preferred_element_type=jnp.float32)

    return pl.pallas_call(
        body,
        out_shape=(jax.ShapeDtypeStruct((N, H * C2 // 2), jnp.float32),
                   jax.ShapeDtypeStruct((N, H * C2 // 2), jnp.float32),
                   jax.ShapeDtypeStruct((N, 2 * H), jnp.float32)),
        name="gat_mid",
    )(op, sp, b1, W2, A2pack, Rexp)


def _tc_final(opa, opb, spa, spb, b2, Rexp):
    def body(opa_ref, opb_ref, spa_ref, spb_ref, b_ref, r_ref, o_ref):
        ou = jnp.concatenate(
            [opa_ref[0] + opa_ref[1], opb_ref[0] + opb_ref[1]], axis=1)
        sv = spa_ref[0] + spa_ref[1] + spb_ref[0] + spb_ref[1]
        scale = jnp.dot(1.0 / (sv + 1e-16), r_ref[...],
                        preferred_element_type=jnp.float32)
        y = ou * scale + b_ref[...]
        m = jnp.max(y, axis=1, keepdims=True)
        z = y - m
        o_ref[...] = z - jnp.log(jnp.sum(jnp.exp(z), axis=1, keepdims=True))

    return pl.pallas_call(
        body,
        out_shape=jax.ShapeDtypeStruct((N, H * C2), jnp.float32),
        name="gat_final",
    )(opa, opb, spa, spb, b2, Rexp)


def _packs(a_src, a_dst, C):
    eye = jnp.eye(H, dtype=jnp.float32)
    Ms = jnp.einsum("hc,hg->hcg", a_src, eye).reshape(H * C, H)
    Md = jnp.einsum("hc,hg->hcg", a_dst, eye).reshape(H * C, H)
    return jnp.concatenate([Ms, Md], axis=1)


def _rexp(C):
    top = jnp.kron(jnp.eye(H, dtype=jnp.float32),
                   jnp.ones((1, C), dtype=jnp.float32))
    return jnp.concatenate(
        [top, jnp.zeros((L - H, H * C), dtype=jnp.float32)], axis=0)


def kernel(x, edge_index, W1, a1_src, a1_dst, b1, W2, a2_src, a2_dst, b2):
    src = edge_index[0].reshape(E // EB, EB)
    dst = edge_index[1].reshape(E // EB, EB)

    h1, p1 = _tc_embed(x, W1, _packs(a1_src, a1_dst, C1))
    op1, sp1 = _sc_layer1(p1, h1, src, dst)
    h2a, h2b, p2 = _tc_mid(op1, sp1, b1.reshape(1, H * C1), W2,
                           _packs(a2_src, a2_dst, C2), _rexp(C1))
    op2a, sp2a = _sc_layer2a(p2, h2a, src, dst)
    op2b, sp2b = _sc_layer2b(p2, h2b, src, dst)
    return _tc_final(op2a, op2b, sp2a, sp2b,
                     b2.reshape(1, H * C2), _rexp(C2))


# speedup vs baseline: 198.6282x; 1.0757x over previous
import functools
import numpy as np
import jax
import jax.numpy as jnp
from jax import lax
from jax.experimental import pallas as pl
from jax.experimental.pallas import tpu as pltpu
from jax.experimental.pallas import tpu_sc as plsc

N = 10000
E = 320000
F_IN = 128
H = 8
C1 = 8
C2 = 16

NC = 2
NS = 16
L = 16
NW = NC * NS
EPT = E // NW
EB = 80
NB = EPT // EB
WB = 624
ZR = 104


def _vgather(x, idx):
    dn = lax.GatherDimensionNumbers(
        offset_dims=(), collapsed_slice_dims=(0,), start_index_map=(0,))
    return lax.gather(x, idx[:, None], dn, (1,),
                      mode=lax.GatherScatterMode.PROMISE_IN_BOUNDS)


def _make_sc_layer(C, HO):
    HS = 64 // C
    HC = HS * C
    NV = HC // L
    HX = HC + L
    mesh = plsc.VectorSubcoreMesh(core_axis_name="c", subcore_axis_name="s")

    def body(p_hbm, h_hbm, src_hbm, dst_hbm, out_hbm,
             src2_v, dst2_v, ps0, ps1, pd0, pd1, hr0, hr1,
             msg0, msg1, z_v,
             out_acc, gsem0, gsem1, ssem0, ssem1):
        c = lax.axis_index("c")
        s = lax.axis_index("s")
        wid = c * NS + s
        lane = lax.iota(jnp.int32, L)
        rot8 = (lane & 7) + 8
        if C == 16:
            scale_idx = [lane * 0 + (HO + v) for v in range(NV)]
        else:
            scale_idx = [2 * v + (lane >> 3) for v in range(NV)]

        bufs = [(ps0, pd0, hr0, msg0, gsem0, ssem0),
                (ps1, pd1, hr1, msg1, gsem1, ssem1)]

        pltpu.sync_copy(src_hbm.at[pl.ds(wid * NB, NB)], src2_v)
        pltpu.sync_copy(dst_hbm.at[pl.ds(wid * NB, NB)], dst2_v)

        def issue_gathers(it, b):
            ps, pd, hr, _, gsem, _ = bufs[b]
            pltpu.async_copy(p_hbm.at[src2_v.at[it]], ps, gsem)
            pltpu.async_copy(p_hbm.at[dst2_v.at[it]], pd, gsem)
            pltpu.async_copy(h_hbm.at[src2_v.at[it]], hr, gsem)

        def drain_gathers(b):
            ps, pd, hr, _, gsem, _ = bufs[b]
            pltpu.make_async_copy(p_hbm.at[pl.ds(0, EB)], ps, gsem).wait()
            pltpu.make_async_copy(p_hbm.at[pl.ds(0, EB)], pd, gsem).wait()
            pltpu.make_async_copy(h_hbm.at[pl.ds(0, EB)], hr, gsem).wait()

        def issue_scatters(it, b):
            _, _, _, msg, _, ssem = bufs[b]
            pltpu.async_copy(msg, out_acc.at[dst2_v.at[it]], ssem, add=True)

        def drain_scatters(b):
            _, _, _, msg, _, ssem = bufs[b]
            pltpu.make_async_copy(msg, out_acc.at[pl.ds(0, EB)], ssem).wait()

        def compute(b):
            ps_v, pd_v, hr_v, msg_v, _, _ = bufs[b]

            @plsc.parallel_loop(0, EB, 1, unroll=4)
            def edge(j):
                ps = ps_v[j]
                pd = pd_v[j]
                e = ps + _vgather(pd, rot8)
                e = jnp.where(e > 0, e, 0.2 * e)
                ex = jnp.exp(e)
                for v in range(NV):
                    sc = _vgather(ex, scale_idx[v])
                    msg_v[j, pl.ds(v * L, L)] = (
                        hr_v[j, pl.ds(v * L, L)] * sc)
                msg_v[j, pl.ds(NV * L, L)] = jnp.where(
                    (lane >= HO) & (lane < HO + HS), ex, 0.0)

        def step(it, b, has_next, drain_scatter):
            if has_next:
                issue_gathers(it + 1, 1 - b)
            drain_gathers(b)
            if drain_scatter:
                drain_scatters(b)
            compute(b)
            issue_scatters(it, b)

        issue_gathers(0, 0)

        def zfill(r, carry):
            for v in range(HX // L):
                z_v[r, pl.ds(v * L, L)] = jnp.zeros((L,), jnp.float32)
            return carry
        lax.fori_loop(0, ZR, zfill, 0)
        row0 = s * WB
        for q in range(WB // ZR):
            pltpu.sync_copy(z_v, out_acc.at[pl.ds(row0 + q * ZR, ZR)])

        @pl.when(s == NS - 1)
        def _():
            pltpu.sync_copy(z_v.at[pl.ds(0, N - NS * WB)],
                            out_acc.at[pl.ds(NS * WB, N - NS * WB)])
        plsc.subcore_barrier()

        step(0, 0, True, False)
        step(1, 1, True, False)

        def pair(k, carry):
            step(2 * k, 0, True, True)
            step(2 * k + 1, 1, True, True)
            return carry
        lax.fori_loop(1, (NB - 1) // 2, pair, 0)
        step(NB - 1, 0, False, True)
        drain_scatters(1)
        drain_scatters(0)
        plsc.subcore_barrier()

        pltpu.sync_copy(out_acc.at[pl.ds(row0, WB)],
                        out_hbm.at[c, pl.ds(row0, WB)])

        @pl.when(s == NS - 1)
        def _():
            pltpu.sync_copy(out_acc.at[pl.ds(NS * WB, N - NS * WB)],
                            out_hbm.at[c, pl.ds(NS * WB, N - NS * WB)])

    return pl.kernel(
        body, mesh=mesh,
        compiler_params=pltpu.CompilerParams(use_tc_tiling_on_sc=False),
        out_type=jax.ShapeDtypeStruct((NC, N, HX), jnp.float32),
        scratch_types=[
            pltpu.VMEM((NB, EB), jnp.int32),
            pltpu.VMEM((NB, EB), jnp.int32),
            pltpu.VMEM((EB, L), jnp.float32),
            pltpu.VMEM((EB, L), jnp.float32),
            pltpu.VMEM((EB, L), jnp.float32),
            pltpu.VMEM((EB, L), jnp.float32),
            pltpu.VMEM((EB, HC), jnp.float32),
            pltpu.VMEM((EB, HC), jnp.float32),
            pltpu.VMEM((EB, HX), jnp.float32),
            pltpu.VMEM((EB, HX), jnp.float32),
            pltpu.VMEM((ZR, HX), jnp.float32),
            pltpu.VMEM_SHARED((N, HX), jnp.float32),
            pltpu.SemaphoreType.DMA,
            pltpu.SemaphoreType.DMA,
            pltpu.SemaphoreType.DMA,
            pltpu.SemaphoreType.DMA,
        ],
        name=f"gat_edges_c{C}h{HO}",
    )


_sc_layer1 = _make_sc_layer(C1, 0)
_sc_layer2a = _make_sc_layer(C2, 0)
_sc_layer2b = _make_sc_layer(C2, 4)


def _tc_embed(x, W, Apack):
    n, hc = x.shape[0], W.shape[1]

    def body(x_ref, w_ref, a_ref, h_ref, p_ref):
        h = jnp.dot(x_ref[...], w_ref[...],
                    preferred_element_type=jnp.float32)
        h_ref[...] = h
        p_ref[...] = jnp.dot(h, a_ref[...],
                             preferred_element_type=jnp.float32)

    return pl.pallas_call(
        body,
        out_shape=(jax.ShapeDtypeStruct((n, hc), jnp.float32),
                   jax.ShapeDtypeStruct((n, 2 * H), jnp.float32)),
        name="gat_embed",
    )(x, W, Apack)


def _tc_mid(op, b1, W2, A2pack, Rexp):
    def body(op_ref, b_ref, w_ref, a_ref, r_ref,
             ha_ref, hb_ref, p_ref):
        acc = op_ref[0] + op_ref[1]
        ou = acc[:, : H * C1]
        sv = acc[:, H * C1:]
        scale = jnp.dot(1.0 / (sv + 1e-16), r_ref[...],
                        preferred_element_type=jnp.float32)
        u = ou * scale + b_ref[...]
        a = jnp.where(u > 0, u, jnp.exp(jnp.minimum(u, 0.0)) - 1.0)
        h2 = jnp.dot(a, w_ref[...], preferred_element_type=jnp.float32)
        ha_ref[...] = h2[:, : H * C2 // 2]
        hb_ref[...] = h2[:, H * C2 // 2:]
        p_ref[...] = jnp.dot(h2, a_ref[...],
                             preferred_element_type=jnp.float32)

    return pl.pallas_call(
        body,
        out_shape=(jax.ShapeDtypeStruct((N, H * C2 // 2), jnp.float32),
                   jax.ShapeDtypeStruct((N, H * C2 // 2), jnp.float32),
                   jax.ShapeDtypeStruct((N, 2 * H), jnp.float32)),
        name="gat_mid",
    )(op, b1, W2, A2pack, Rexp)


def _tc_final(opa, opb, b2, Rexp):
    def body(opa_ref, opb_ref, b_ref, r_ref, o_ref):
        acca = opa_ref[0] + opa_ref[1]
        accb = opb_ref[0] + opb_ref[1]
        HW = H * C2 // 2
        ou = jnp.concatenate([acca[:, :HW], accb[:, :HW]], axis=1)
        sv = acca[:, HW:] + accb[:, HW:]
        scale = jnp.dot(1.0 / (sv + 1e-16), r_ref[...],
                        preferred_element_type=jnp.float32)
        y = ou * scale + b_ref[...]
        m = jnp.max(y, axis=1, keepdims=True)
        z = y - m
        o_ref[...] = z - jnp.log(jnp.sum(jnp.exp(z), axis=1, keepdims=True))

    return pl.pallas_call(
        body,
        out_shape=jax.ShapeDtypeStruct((N, H * C2), jnp.float32),
        name="gat_final",
    )(opa, opb, b2, Rexp)


def _packs(a_src, a_dst, C):
    eye = jnp.eye(H, dtype=jnp.float32)
    Ms = jnp.einsum("hc,hg->hcg", a_src, eye).reshape(H * C, H)
    Md = jnp.einsum("hc,hg->hcg", a_dst, eye).reshape(H * C, H)
    return jnp.concatenate([Ms, Md], axis=1)


def _rexp(C):
    top = jnp.kron(jnp.eye(H, dtype=jnp.float32),
                   jnp.ones((1, C), dtype=jnp.float32))
    return jnp.concatenate(
        [top, jnp.zeros((L - H, H * C), dtype=jnp.float32)], axis=0)


def kernel(x, edge_index, W1, a1_src, a1_dst, b1, W2, a2_src, a2_dst, b2):
    src = edge_index[0].reshape(E // EB, EB)
    dst = edge_index[1].reshape(E // EB, EB)

    h1, p1 = _tc_embed(x, W1, _packs(a1_src, a1_dst, C1))
    op1 = _sc_layer1(p1, h1, src, dst)
    h2a, h2b, p2 = _tc_mid(op1, b1.reshape(1, H * C1), W2,
                           _packs(a2_src, a2_dst, C2), _rexp(C1))
    op2a = _sc_layer2a(p2, h2a, src, dst)
    op2b = _sc_layer2b(p2, h2b, src, dst)
    return _tc_final(op2a, op2b, b2.reshape(1, H * C2), _rexp(C2))
